# baseline (speedup 1.0000x reference)
import jax
import jax.numpy as jnp
from jax.experimental import pallas as pl
from jax.experimental.pallas import tpu as pltpu

_TM = 1024
_RESIDENT = 24576
_FULL_STEPS = 4


def _hybrid_kernel(idx_ref, w_hbm, out_ref, table_ref, row_sem, tbl_sem,
                   *, inner_steps, resident, full_steps):
    c = pl.program_id(0)
    j = pl.program_id(1)
    tm = out_ref.shape[0]
    base = (c * inner_steps + j) * tm

    @pl.when(j == 0)
    def _():
        pltpu.make_async_copy(
            w_hbm.at[pl.ds(0, resident)], table_ref, tbl_sem
        ).start(priority=1)

    @pl.when(j < full_steps)
    def _():
        for r in range(tm):
            row = idx_ref[base + r]
            pltpu.make_async_copy(
                w_hbm.at[pl.ds(row, 1)],
                out_ref.at[pl.ds(r, 1)],
                row_sem,
            ).start()
        pltpu.make_async_copy(
            w_hbm.at[pl.ds(0, tm)],
            out_ref.at[pl.ds(0, tm)],
            row_sem,
        ).wait()

    @pl.when(j == full_steps - 1)
    def _():
        pltpu.make_async_copy(
            w_hbm.at[pl.ds(0, resident)], table_ref, tbl_sem
        ).wait()

    @pl.when(j >= full_steps)
    def _():
        for r in range(tm):
            row = idx_ref[base + r]
            res = jnp.minimum(row, resident - 1)
            out_ref[r] = table_ref[res]
        cnt = jnp.int32(0)
        for r in range(tm):
            row = idx_ref[base + r]
            miss = row >= resident

            @pl.when(miss)
            def _issue():
                pltpu.make_async_copy(
                    w_hbm.at[pl.ds(row, 1)],
                    out_ref.at[pl.ds(r, 1)],
                    row_sem,
                ).start(priority=r & 1)

            cnt = cnt + miss.astype(jnp.int32)

        @pl.when(cnt > 0)
        def _drain():
            pltpu.make_async_copy(
                w_hbm.at[pl.ds(0, cnt)],
                out_ref.at[pl.ds(0, cnt)],
                row_sem,
            ).wait()


def _gather_kernel(idx_ref, w_hbm, out_ref, sem):
    tm = out_ref.shape[0]
    base = pl.program_id(0) * tm
    for r in range(tm):
        row = idx_ref[base + r]
        pltpu.make_async_copy(
            w_hbm.at[pl.ds(row, 1), :],
            out_ref.at[pl.ds(r, 1), :],
            sem,
        ).start(priority=r & 1)
    pltpu.make_async_copy(
        w_hbm.at[pl.ds(0, tm), :],
        out_ref.at[pl.ds(0, tm), :],
        sem,
    ).wait()


def _pure_dma(flat_idx, weight, n):
    num_embeddings, embedding_dim = weight.shape
    tile = 4096
    tm = tile if n % tile == 0 else (n if n <= tile else 8)
    n_pad = -(-n // tm) * tm
    if n_pad != n:
        flat_idx = jnp.pad(flat_idx, (0, n_pad - n))
    grid_spec = pltpu.PrefetchScalarGridSpec(
        num_scalar_prefetch=1,
        grid=(n_pad // tm,),
        in_specs=[pl.BlockSpec(memory_space=pl.ANY)],
        out_specs=pl.BlockSpec((tm, embedding_dim), lambda i, idx: (i, 0)),
        scratch_shapes=[pltpu.SemaphoreType.DMA],
    )
    flat_out = pl.pallas_call(
        _gather_kernel,
        out_shape=jax.ShapeDtypeStruct((n_pad, embedding_dim), weight.dtype),
        grid_spec=grid_spec,
        compiler_params=pltpu.CompilerParams(
            dimension_semantics=("parallel",),
            disable_bounds_checks=True,
        ),
    )(flat_idx, weight)
    return flat_out[:n] if n_pad != n else flat_out


def kernel(indices, weight):
    import functools

    num_embeddings, embedding_dim = weight.shape
    orig_shape = indices.shape
    flat_idx = indices.reshape(-1)
    if flat_idx.dtype != jnp.int32:
        flat_idx = flat_idx.astype(jnp.int32)
    n = flat_idx.shape[0]
    if n == 0:
        return jnp.zeros(orig_shape + (embedding_dim,), weight.dtype)

    resident = _RESIDENT
    tiles = n // _TM
    if (n % (2 * _TM) or num_embeddings <= resident
            or tiles // 2 <= _FULL_STEPS):
        flat_out = _pure_dma(flat_idx, weight, n)
        return flat_out.reshape(orig_shape + (embedding_dim,))

    inner_steps = tiles // 2
    w3 = weight.reshape(num_embeddings, 1, embedding_dim)
    grid_spec = pltpu.PrefetchScalarGridSpec(
        num_scalar_prefetch=1,
        grid=(2, inner_steps),
        in_specs=[pl.BlockSpec(memory_space=pl.ANY)],
        out_specs=pl.BlockSpec(
            (_TM, 1, embedding_dim),
            lambda c, j, idx: (c * inner_steps + j, 0, 0),
        ),
        scratch_shapes=[
            pltpu.VMEM((resident, 1, embedding_dim), weight.dtype),
            pltpu.SemaphoreType.DMA,
            pltpu.SemaphoreType.DMA,
        ],
    )
    flat_out = pl.pallas_call(
        functools.partial(
            _hybrid_kernel,
            inner_steps=inner_steps,
            resident=resident,
            full_steps=_FULL_STEPS,
        ),
        out_shape=jax.ShapeDtypeStruct((n, 1, embedding_dim), weight.dtype),
        grid_spec=grid_spec,
        compiler_params=pltpu.CompilerParams(
            dimension_semantics=("parallel", "arbitrary"),
            disable_bounds_checks=True,
        ),
    )(flat_idx, w3)
    return flat_out.reshape(orig_shape + (embedding_dim,))


# speedup vs baseline: 1.0308x; 1.0308x over previous
import jax
import jax.numpy as jnp
from jax.experimental import pallas as pl
from jax.experimental.pallas import tpu as pltpu

_TM = 1024
_RESIDENT = 24576
_FULL_STEPS = 7


def _hybrid_kernel(idx_ref, w_hbm, out_ref, table_ref, row_sem, tbl_sem,
                   *, inner_steps, resident, full_steps):
    c = pl.program_id(0)
    j = pl.program_id(1)
    tm = out_ref.shape[0]
    base = (c * inner_steps + j) * tm

    @pl.when(j == 0)
    def _():
        pltpu.make_async_copy(
            w_hbm.at[pl.ds(0, resident)], table_ref, tbl_sem
        ).start(priority=1)

    @pl.when(j < full_steps)
    def _():
        for r in range(tm):
            row = idx_ref[base + r]
            pltpu.make_async_copy(
                w_hbm.at[pl.ds(row, 1)],
                out_ref.at[pl.ds(r, 1)],
                row_sem,
            ).start()
        pltpu.make_async_copy(
            w_hbm.at[pl.ds(0, tm)],
            out_ref.at[pl.ds(0, tm)],
            row_sem,
        ).wait()

    @pl.when(j == full_steps - 1)
    def _():
        pltpu.make_async_copy(
            w_hbm.at[pl.ds(0, resident)], table_ref, tbl_sem
        ).wait()

    @pl.when(j >= full_steps)
    def _():
        for r in range(tm):
            row = idx_ref[base + r]
            res = jnp.minimum(row, resident - 1)
            out_ref[r] = table_ref[res]
        cnt = jnp.int32(0)
        for r in range(tm):
            row = idx_ref[base + r]
            miss = row >= resident

            @pl.when(miss)
            def _issue():
                pltpu.make_async_copy(
                    w_hbm.at[pl.ds(row, 1)],
                    out_ref.at[pl.ds(r, 1)],
                    row_sem,
                ).start(priority=r & 1)

            cnt = cnt + miss.astype(jnp.int32)

        @pl.when(cnt > 0)
        def _drain():
            pltpu.make_async_copy(
                w_hbm.at[pl.ds(0, cnt)],
                out_ref.at[pl.ds(0, cnt)],
                row_sem,
            ).wait()


def _gather_kernel(idx_ref, w_hbm, out_ref, sem):
    tm = out_ref.shape[0]
    base = pl.program_id(0) * tm
    for r in range(tm):
        row = idx_ref[base + r]
        pltpu.make_async_copy(
            w_hbm.at[pl.ds(row, 1), :],
            out_ref.at[pl.ds(r, 1), :],
            sem,
        ).start(priority=r & 1)
    pltpu.make_async_copy(
        w_hbm.at[pl.ds(0, tm), :],
        out_ref.at[pl.ds(0, tm), :],
        sem,
    ).wait()


def _pure_dma(flat_idx, weight, n):
    num_embeddings, embedding_dim = weight.shape
    tile = 4096
    tm = tile if n % tile == 0 else (n if n <= tile else 8)
    n_pad = -(-n // tm) * tm
    if n_pad != n:
        flat_idx = jnp.pad(flat_idx, (0, n_pad - n))
    grid_spec = pltpu.PrefetchScalarGridSpec(
        num_scalar_prefetch=1,
        grid=(n_pad // tm,),
        in_specs=[pl.BlockSpec(memory_space=pl.ANY)],
        out_specs=pl.BlockSpec((tm, embedding_dim), lambda i, idx: (i, 0)),
        scratch_shapes=[pltpu.SemaphoreType.DMA],
    )
    flat_out = pl.pallas_call(
        _gather_kernel,
        out_shape=jax.ShapeDtypeStruct((n_pad, embedding_dim), weight.dtype),
        grid_spec=grid_spec,
        compiler_params=pltpu.CompilerParams(
            dimension_semantics=("parallel",),
            disable_bounds_checks=True,
        ),
    )(flat_idx, weight)
    return flat_out[:n] if n_pad != n else flat_out


def kernel(indices, weight):
    import functools

    num_embeddings, embedding_dim = weight.shape
    orig_shape = indices.shape
    flat_idx = indices.reshape(-1)
    if flat_idx.dtype != jnp.int32:
        flat_idx = flat_idx.astype(jnp.int32)
    n = flat_idx.shape[0]
    if n == 0:
        return jnp.zeros(orig_shape + (embedding_dim,), weight.dtype)

    resident = _RESIDENT
    tiles = n // _TM
    if (n % (2 * _TM) or num_embeddings <= resident
            or tiles // 2 <= _FULL_STEPS):
        flat_out = _pure_dma(flat_idx, weight, n)
        return flat_out.reshape(orig_shape + (embedding_dim,))

    inner_steps = tiles // 2
    w3 = weight.reshape(num_embeddings, 1, embedding_dim)
    grid_spec = pltpu.PrefetchScalarGridSpec(
        num_scalar_prefetch=1,
        grid=(2, inner_steps),
        in_specs=[pl.BlockSpec(memory_space=pl.ANY)],
        out_specs=pl.BlockSpec(
            (_TM, 1, embedding_dim),
            lambda c, j, idx: (c * inner_steps + j, 0, 0),
        ),
        scratch_shapes=[
            pltpu.VMEM((resident, 1, embedding_dim), weight.dtype),
            pltpu.SemaphoreType.DMA,
            pltpu.SemaphoreType.DMA,
        ],
    )
    flat_out = pl.pallas_call(
        functools.partial(
            _hybrid_kernel,
            inner_steps=inner_steps,
            resident=resident,
            full_steps=_FULL_STEPS,
        ),
        out_shape=jax.ShapeDtypeStruct((n, 1, embedding_dim), weight.dtype),
        grid_spec=grid_spec,
        compiler_params=pltpu.CompilerParams(
            dimension_semantics=("parallel", "arbitrary"),
            disable_bounds_checks=True,
        ),
    )(flat_idx, w3)
    return flat_out.reshape(orig_shape + (embedding_dim,))
